# baseline (speedup 1.0000x reference)
import functools

import jax
import jax.numpy as jnp
from jax import lax
from jax.experimental import pallas as pl
from jax.experimental.pallas import tpu as pltpu
from jax.experimental.pallas import tpu_sc as plsc

_VOCAB = 1000000
_EMBED = 300
_BATCH = 1024
_SEQ = 512
_HIDDEN = 16

_NC, _NS = 2, 16
_NW = _NC * _NS
_RPW = _BATCH // _NW
_CHUNK = 128
_NCH = _SEQ // _CHUNK
_NG = _RPW * _NCH
_BAND = 1000
_NBAND = 128 // _HIDDEN
_PBLK = _BAND * _NBAND
_P2ROWS = _VOCAB // _NBAND


_TROWS = _BATCH * _SEQ // 128


def _proj_body(x_ref, w_ref, t_ref, o_ref, row_ref, colb_ref):
    acc = jnp.zeros((_BAND, 128), jnp.float32)
    for j in range(_NBAND):
        xj = x_ref[pl.ds(j * _BAND, _BAND), :]
        wj = w_ref[pl.ds(j * _EMBED, _EMBED), :]
        acc = acc + jnp.dot(xj.astype(jnp.bfloat16), wj,
                            preferred_element_type=jnp.float32)
    o_ref[:] = acc

    @pl.when(pl.program_id(0) == 0)
    def _():
        t = t_ref[:]
        blk = t // _PBLK
        band = (t // _BAND) % _NBAND
        r = t % _BAND
        row_ref[:] = blk * _BAND + r
        colb_ref[:] = band * _HIDDEN


_proj = pl.pallas_call(
    _proj_body,
    grid=(_VOCAB // _PBLK,),
    in_specs=[
        pl.BlockSpec((_PBLK, _EMBED), lambda i: (i, 0)),
        pl.BlockSpec((_NBAND * _EMBED, 128), lambda i: (0, 0)),
        pl.BlockSpec((_TROWS, 128), lambda i: (0, 0)),
    ],
    out_specs=[
        pl.BlockSpec((_BAND, 128), lambda i: (i, 0)),
        pl.BlockSpec((_TROWS, 128), lambda i: (0, 0)),
        pl.BlockSpec((_TROWS, 128), lambda i: (0, 0)),
    ],
    out_shape=[
        jax.ShapeDtypeStruct((_P2ROWS, 128), jnp.float32),
        jax.ShapeDtypeStruct((_TROWS, 128), jnp.int32),
        jax.ShapeDtypeStruct((_TROWS, 128), jnp.int32),
    ],
)


_NBUF = 4


def _bag_body(row_hbm, colb_hbm, p2_hbm, b1_hbm, w2b_hbm, b2_hbm, out_hbm,
              row_v, colb_v, rows0_v, rows1_v, rows2_v, rows3_v, tr_v,
              b1_v, w2b_v, b2_v, hsT_v, out_v,
              sem0, sem1, sem2, sem3):
    wid = lax.axis_index("s") * _NC + lax.axis_index("c")
    tpw = _RPW * _SEQ
    pltpu.sync_copy(row_hbm.at[pl.ds(wid * tpw, tpw)], row_v)
    pltpu.sync_copy(colb_hbm.at[pl.ds(wid * tpw, tpw)], colb_v)
    pltpu.sync_copy(b1_hbm, b1_v)
    pltpu.sync_copy(w2b_hbm, w2b_v)
    pltpu.sync_copy(b2_hbm, b2_v)

    sems = (sem0, sem1, sem2, sem3)
    bufs = (rows0_v, rows1_v, rows2_v, rows3_v)

    def gather(g, slot):
        idx = row_v.at[pl.ds(g * _CHUNK, _CHUNK)]
        return pltpu.async_copy(p2_hbm.at[idx], bufs[slot], sems[slot])

    def gather_wait(g, slot):
        idx = row_v.at[pl.ds(g * _CHUNK, _CHUNK)]
        pltpu.make_async_copy(p2_hbm.at[idx], bufs[slot], sems[slot]).wait()

    for s in range(_NBUF):
        gather(s, s)

    lane = lax.iota(jnp.int32, 16)

    def row_body(r, _):
        accT = (jnp.zeros((16,), jnp.float32),) * 16
        for c in range(_NCH):
            g = r * _NCH + c
            slot = c % _NBUF
            gather_wait(g, slot)

            @pl.when(g + _NBUF < _NG)
            def _():
                gather(g + _NBUF, slot)

            buf = bufs[slot]

            def group_body(gi, a):
                off = g * _CHUNK + gi * 16
                colb = colb_v[pl.ds(off, 16)]
                rowi = lane + gi * 16
                return tuple(
                    a[l] + plsc.load_gather(buf, [rowi, colb + l])
                    for l in range(16)
                )

            accT = lax.fori_loop(0, _CHUNK // 16, group_body, accT)

        for l in range(16):
            plsc.store_scatter(tr_v, [lane, jnp.full((16,), l, jnp.int32)],
                               accT[l])
        pooled = tr_v[0, :]
        for k in range(1, 16):
            pooled = pooled + tr_v[k, :]

        h = jnp.maximum(pooled + b1_v[pl.ds(0, 16)], 0.0)
        rc = jnp.bitwise_and(r, 15)
        plsc.store_scatter(hsT_v, [lane, lane * 0 + rc], h)

        @pl.when(rc == 15)
        def _():
            z = b2_v[pl.ds(0, 16)]
            for k in range(16):
                z = z + hsT_v[k, :] * w2b_v[k, pl.ds(0, 16)]
            sig = 1.0 / (1.0 + jnp.exp(-z))
            gidx = lax.shift_right_logical(r, 4)
            out_v[pl.ds(gidx * 16, 16)] = sig

        return 0

    lax.fori_loop(0, _RPW, row_body, 0)
    pltpu.sync_copy(out_v, out_hbm.at[pl.ds(wid * _RPW, _RPW)])


_bag = functools.partial(
    pl.kernel,
    out_type=jax.ShapeDtypeStruct((_BATCH,), jnp.float32),
    mesh=plsc.VectorSubcoreMesh(core_axis_name="c", subcore_axis_name="s"),
    scratch_types=[
        pltpu.VMEM((_RPW * _SEQ,), jnp.int32),
        pltpu.VMEM((_RPW * _SEQ,), jnp.int32),
        pltpu.VMEM((_CHUNK, 128), jnp.float32),
        pltpu.VMEM((_CHUNK, 128), jnp.float32),
        pltpu.VMEM((_CHUNK, 128), jnp.float32),
        pltpu.VMEM((_CHUNK, 128), jnp.float32),
        pltpu.VMEM((16, 16), jnp.float32),
        pltpu.VMEM((16,), jnp.float32),
        pltpu.VMEM((16, 16), jnp.float32),
        pltpu.VMEM((16,), jnp.float32),
        pltpu.VMEM((16, 16), jnp.float32),
        pltpu.VMEM((_RPW,), jnp.float32),
        pltpu.SemaphoreType.DMA,
        pltpu.SemaphoreType.DMA,
        pltpu.SemaphoreType.DMA,
        pltpu.SemaphoreType.DMA,
    ],
    compiler_params=pltpu.CompilerParams(needs_layout_passes=False),
)(_bag_body)


def kernel(tokens, emb_table, W1, b1, W2, b2):
    w1s = W1 * jnp.float32(1.0 / _SEQ)
    w1bd = jnp.zeros((_NBAND * _EMBED, 128), jnp.float32)
    for j in range(_NBAND):
        w1bd = w1bd.at[j * _EMBED:(j + 1) * _EMBED,
                       j * _HIDDEN:(j + 1) * _HIDDEN].set(w1s)
    w1bd = w1bd.astype(jnp.bfloat16)

    p2, grow, gcolb = _proj(emb_table, w1bd, tokens.reshape(-1, 128))
    w2b = jnp.broadcast_to(W2, (_HIDDEN, _HIDDEN))
    b2b = jnp.broadcast_to(b2, (_HIDDEN,))
    out_flat = _bag(grow.reshape(-1), gcolb.reshape(-1), p2, b1, w2b, b2b)
    return out_flat.reshape(_BATCH, 1)
